# baseline (speedup 1.0000x reference)
import functools

import jax
import jax.numpy as jnp
from jax import lax
from jax.experimental import pallas as pl
from jax.experimental.pallas import tpu as pltpu
from jax.experimental.pallas import tpu_sc as plsc

N = 10000
E = 320000
D = 128
H = D // 2
G = 64

_HI = lax.Precision.HIGHEST
_DP = lax.Precision.DEFAULT

def _proj_body(x_ref, ws_ref, wd_ref, xs_ref, xd_ref):
    x = x_ref[...]
    xs_ref[...] = jnp.dot(x, ws_ref[...], precision=_DP)
    xd_ref[...] = jnp.dot(x, wd_ref[...], precision=_DP)


def _node_proj(x, ws, wd):
    return pl.pallas_call(
        _proj_body,
        out_shape=[jax.ShapeDtypeStruct((N, D), jnp.float32),
                   jax.ShapeDtypeStruct((N, D), jnp.float32)],
    )(x, ws, wd)


_NW = 32
_BLK = 80
_NBT = E // _BLK
_KMAX = (_NBT + _NW - 1) // _NW
_RPS = 624
_TAIL = N - 16 * _RPS


def _sc_body(xs_hbm, xd_hbm, ea_hbm, we_hbm, src_hbm, dst_hbm, zero_hbm,
             out_hbm, agg_sh, wv, srcv0, dstv0, srcv1, dstv1, bufa0, bufb0,
             bufa1, bufb1, eav0, eav1, sa0, sb0, sa1, sb1, se0, se1):
    c = lax.axis_index("c")
    s = lax.axis_index("s")
    wid = c * 16 + s
    rows = pl.ds(s * _RPS, _RPS)
    tail = pl.ds(16 * _RPS, _TAIL)
    pltpu.sync_copy(zero_hbm.at[rows], agg_sh.at[rows])

    @pl.when(s == 0)
    def _zero_tail():
        pltpu.sync_copy(zero_hbm.at[tail], agg_sh.at[tail])

    pltpu.sync_copy(we_hbm, wv)
    plsc.subcore_barrier()
    wvals = [[wv[k, pl.ds(r * 16, 16)] for r in range(D // 16)]
             for k in range(4)]

    bufs = ((srcv0, dstv0, bufa0, bufb0, eav0, sa0, sb0, se0),
            (srcv1, dstv1, bufa1, bufb1, eav1, sa1, sb1, se1))

    def start(k, p):
        srcv, dstv, bufa, bufb, eav, sa, sb, se = bufs[p]
        b = k * _NW + wid

        @pl.when(b < _NBT)
        def _():
            base = b * _BLK
            pltpu.sync_copy(src_hbm.at[pl.ds(base, _BLK)], srcv)
            pltpu.sync_copy(dst_hbm.at[pl.ds(base, _BLK)], dstv)
            pltpu.async_copy(xs_hbm.at[srcv], bufa, sa)
            pltpu.async_copy(xd_hbm.at[dstv], bufb, sb)
            pltpu.async_copy(
                ea_hbm.at[pl.ds(base * 4, _BLK * 4)], eav, se)

    def finish(k, p):
        srcv, dstv, bufa, bufb, eav, sa, sb, se = bufs[p]
        b = k * _NW + wid

        @pl.when(b < _NBT)
        def _():
            base = b * _BLK
            pltpu.make_async_copy(xs_hbm.at[srcv], bufa, sa).wait()
            pltpu.make_async_copy(xd_hbm.at[dstv], bufb, sb).wait()
            pltpu.make_async_copy(
                ea_hbm.at[pl.ds(base * 4, _BLK * 4)], eav, se).wait()

            def grp(g, _):
                eavec = eav[pl.ds(g * 16, 16)]
                for jj in range(4):
                    j = g * 4 + jj
                    e = [lax.gather(
                            eavec,
                            jnp.full((16, 1), 4 * jj + k2, jnp.int32),
                            lax.GatherDimensionNumbers(
                                offset_dims=(),
                                collapsed_slice_dims=(0,),
                                start_index_map=(0,)),
                            (1,),
                            mode=lax.GatherScatterMode.PROMISE_IN_BOUNDS)
                         for k2 in range(4)]
                    for r in range(D // 16):
                        sl = pl.ds(r * 16, 16)
                        v = bufa[j, sl] + bufb[j, sl]
                        for k2 in range(4):
                            v += e[k2] * wvals[k2][r]
                        bufa[j, sl] = jnp.maximum(v, 0.0)
                return 0

            lax.fori_loop(0, _BLK // 4, grp, 0)
            pltpu.sync_copy(bufa, agg_sh.at[dstv], add=True)

    start(0, 0)

    def pair(kk, carry):
        ka = 2 * kk + 1
        start(ka, 1)
        finish(2 * kk, 0)
        start(ka + 1, 0)
        finish(ka, 1)
        return carry

    lax.fori_loop(0, (_KMAX - 1) // 2, pair, 0)
    finish(_KMAX - 1, 0)

    plsc.subcore_barrier()
    pltpu.sync_copy(agg_sh.at[rows], out_hbm.at[c, rows])

    @pl.when(s == 0)
    def _out_tail():
        pltpu.sync_copy(agg_sh.at[tail], out_hbm.at[c, tail])


def _sc_msgpass(xs, xd, ea_flat, we, src, dst, zeros):
    mesh = plsc.VectorSubcoreMesh(core_axis_name="c", subcore_axis_name="s")
    f = functools.partial(
        pl.kernel,
        mesh=mesh,
        out_type=jax.ShapeDtypeStruct((2, N, D), jnp.float32),
        scratch_types=[
            pltpu.VMEM_SHARED((N, D), jnp.float32),
            pltpu.VMEM((4, D), jnp.float32),
            pltpu.VMEM((_BLK,), jnp.int32),
            pltpu.VMEM((_BLK,), jnp.int32),
            pltpu.VMEM((_BLK,), jnp.int32),
            pltpu.VMEM((_BLK,), jnp.int32),
            pltpu.VMEM((_BLK, D), jnp.float32),
            pltpu.VMEM((_BLK, D), jnp.float32),
            pltpu.VMEM((_BLK, D), jnp.float32),
            pltpu.VMEM((_BLK, D), jnp.float32),
            pltpu.VMEM((_BLK * 4,), jnp.float32),
            pltpu.VMEM((_BLK * 4,), jnp.float32),
            pltpu.SemaphoreType.DMA,
            pltpu.SemaphoreType.DMA,
            pltpu.SemaphoreType.DMA,
            pltpu.SemaphoreType.DMA,
            pltpu.SemaphoreType.DMA,
            pltpu.SemaphoreType.DMA,
        ],
    )(_sc_body)
    return f(xs, xd, ea_flat, we, src, dst, zeros)


_BN = 1000
_NBN = N // _BN


def _post_body(x_ref, aggp_ref, ids_ref, wu_ref, wd_ref, wt_ref,
               out_ref, cont_ref, sums_s, counts_s):
    i = pl.program_id(0)

    @pl.when(i == 0)
    def _init():
        sums_s[...] = jnp.zeros_like(sums_s)
        counts_s[...] = jnp.zeros_like(counts_s)

    x = x_ref[...]
    agg = aggp_ref[0] + aggp_ref[1]
    new_lat = jax.nn.relu(
        jnp.dot(x, wu_ref[:D, :], precision=_DP)
        + jnp.dot(agg, wu_ref[D:, :], precision=_DP))
    out_ref[...] = (jnp.dot(x, wd_ref[:D, :], precision=_DP)
                    + jnp.dot(new_lat, wd_ref[D:, :], precision=_DP))
    ids = ids_ref[0]
    iota = lax.broadcasted_iota(jnp.int32, (G, _BN), 0)
    onehot = (ids == iota).astype(jnp.float32)
    sums_s[...] += lax.dot_general(
        onehot, new_lat, dimension_numbers=(((1,), (0,)), ((), ())),
        precision=_HI)
    counts_s[...] += jnp.sum(onehot, axis=1, keepdims=True)

    @pl.when(i == _NBN - 1)
    def _fin():
        graph_emb = sums_s[...] / jnp.maximum(counts_s[...], 1.0)
        cont_ref[...] = jnp.dot(graph_emb, wt_ref[...], precision=_DP)


def _post(x, aggp, ids_row, w_upd, w_dec, w_term):
    return pl.pallas_call(
        _post_body,
        grid=(_NBN,),
        in_specs=[pl.BlockSpec((_BN, D), lambda i: (i, 0)),
                  pl.BlockSpec((2, _BN, D), lambda i: (0, i, 0)),
                  pl.BlockSpec((1, 1, _BN), lambda i: (i, 0, 0)),
                  pl.BlockSpec((2 * D, D), lambda i: (0, 0)),
                  pl.BlockSpec((2 * D, 1), lambda i: (0, 0)),
                  pl.BlockSpec((D, 1), lambda i: (0, 0))],
        out_specs=[pl.BlockSpec((_BN, 1), lambda i: (i, 0)),
                   pl.BlockSpec((G, 1), lambda i: (0, 0))],
        out_shape=[jax.ShapeDtypeStruct((N, 1), jnp.float32),
                   jax.ShapeDtypeStruct((G, 1), jnp.float32)],
        scratch_shapes=[pltpu.VMEM((G, D), jnp.float32),
                        pltpu.VMEM((G, 1), jnp.float32)],
    )(x, aggp, ids_row, w_upd, w_dec, w_term)


def kernel(x, edge_index, edge_attr, batch_ids, W_msg, W_upd, W_dec, W_term):
    src = edge_index[0]
    dst = edge_index[1]
    ws = W_msg[:D]
    wd = W_msg[D:2 * D]
    we = W_msg[2 * D:].astype(jnp.bfloat16).astype(jnp.float32)
    ea_flat = edge_attr.astype(jnp.bfloat16).astype(jnp.float32).reshape(-1)
    zeros = jnp.zeros((N, D), jnp.float32)
    ids_row = batch_ids.astype(jnp.int32).reshape(_NBN, 1, _BN)

    xs, xd = _node_proj(x, ws, wd)
    aggp = _sc_msgpass(xs, xd, ea_flat, we, src, dst, zeros)
    out, cont = _post(x, aggp, ids_row, W_upd, W_dec, W_term)
    return out, cont.reshape(-1)


# speedup vs baseline: 1.1792x; 1.1792x over previous
import functools

import jax
import jax.numpy as jnp
from jax import lax
from jax.experimental import pallas as pl
from jax.experimental.pallas import tpu as pltpu
from jax.experimental.pallas import tpu_sc as plsc

N = 10000
E = 320000
D = 128
G = 64

_HI = lax.Precision.HIGHEST
_DP = lax.Precision.DEFAULT

def _proj_body(x_ref, ws_ref, wd_ref, xs_ref, xd_ref):
    x = x_ref[...]
    xs_ref[...] = jnp.dot(x, ws_ref[...], precision=_DP)
    xd_ref[...] = jnp.dot(x, wd_ref[...], precision=_DP)


def _node_proj(x, ws, wd):
    return pl.pallas_call(
        _proj_body,
        out_shape=[jax.ShapeDtypeStruct((N, D), jnp.float32),
                   jax.ShapeDtypeStruct((N, D), jnp.float32)],
    )(x, ws, wd)


_BE = 3200
_NBE = E // _BE


def _et_body(ea_ref, we_ref, et_ref):
    et_ref[...] = jnp.dot(ea_ref[...], we_ref[...], precision=_DP)


def _edge_term(ea, we):
    return pl.pallas_call(
        _et_body,
        grid=(_NBE,),
        in_specs=[pl.BlockSpec((_BE, 4), lambda i: (i, 0)),
                  pl.BlockSpec((4, D), lambda i: (0, 0))],
        out_specs=pl.BlockSpec((_BE, D), lambda i: (i, 0)),
        out_shape=jax.ShapeDtypeStruct((E, D), jnp.float32),
    )(ea, we)


_NW = 32
_BLK = 64
_NBT = E // _BLK
_KMAX = (_NBT + _NW - 1) // _NW
_RPS = 624
_TAIL = N - 16 * _RPS


def _sc_body(xs_hbm, xd_hbm, et_hbm, src_hbm, dst_hbm, zero_hbm, out_hbm,
             agg_sh, srcv0, dstv0, srcv1, dstv1, bufa0, bufb0, bufa1, bufb1,
             eta0, eta1, sa0, sb0, sa1, sb1, se0, se1):
    c = lax.axis_index("c")
    s = lax.axis_index("s")
    wid = c * 16 + s
    rows = pl.ds(s * _RPS, _RPS)
    tail = pl.ds(16 * _RPS, _TAIL)
    pltpu.sync_copy(zero_hbm.at[rows], agg_sh.at[rows])

    @pl.when(s == 0)
    def _zero_tail():
        pltpu.sync_copy(zero_hbm.at[tail], agg_sh.at[tail])

    plsc.subcore_barrier()

    bufs = ((srcv0, dstv0, bufa0, bufb0, eta0, sa0, sb0, se0),
            (srcv1, dstv1, bufa1, bufb1, eta1, sa1, sb1, se1))

    def start(k, p):
        srcv, dstv, bufa, bufb, eta, sa, sb, se = bufs[p]
        b = k * _NW + wid

        @pl.when(b < _NBT)
        def _():
            base = b * _BLK
            pltpu.sync_copy(src_hbm.at[pl.ds(base, _BLK)], srcv)
            pltpu.sync_copy(dst_hbm.at[pl.ds(base, _BLK)], dstv)
            pltpu.async_copy(xs_hbm.at[srcv], bufa, sa)
            pltpu.async_copy(xd_hbm.at[dstv], bufb, sb)
            pltpu.async_copy(et_hbm.at[pl.ds(base, _BLK)], eta, se)

    def finish(k, p):
        srcv, dstv, bufa, bufb, eta, sa, sb, se = bufs[p]
        b = k * _NW + wid

        @pl.when(b < _NBT)
        def _():
            base = b * _BLK
            pltpu.make_async_copy(xs_hbm.at[srcv], bufa, sa).wait()
            pltpu.make_async_copy(xd_hbm.at[dstv], bufb, sb).wait()
            pltpu.make_async_copy(
                et_hbm.at[pl.ds(base, _BLK)], eta, se).wait()

            def row(j, _):
                for r in range(D // 16):
                    sl = pl.ds(r * 16, 16)
                    v = eta[j, sl] + bufa[j, sl] + bufb[j, sl]
                    eta[j, sl] = jnp.maximum(v, 0.0)
                return 0

            lax.fori_loop(0, _BLK, row, 0)
            pltpu.sync_copy(eta, agg_sh.at[dstv], add=True)

    start(0, 0)

    def pair(kk, carry):
        ka = 2 * kk + 1
        start(ka, 1)
        finish(2 * kk, 0)
        start(ka + 1, 0)
        finish(ka, 1)
        return carry

    lax.fori_loop(0, (_KMAX - 1) // 2, pair, 0)
    finish(_KMAX - 1, 0)

    plsc.subcore_barrier()
    pltpu.sync_copy(agg_sh.at[rows], out_hbm.at[c, rows])

    @pl.when(s == 0)
    def _out_tail():
        pltpu.sync_copy(agg_sh.at[tail], out_hbm.at[c, tail])


def _sc_msgpass(xs, xd, et, src, dst, zeros):
    mesh = plsc.VectorSubcoreMesh(core_axis_name="c", subcore_axis_name="s")
    f = functools.partial(
        pl.kernel,
        mesh=mesh,
        out_type=jax.ShapeDtypeStruct((2, N, D), jnp.float32),
        scratch_types=[
            pltpu.VMEM_SHARED((N, D), jnp.float32),
            pltpu.VMEM((_BLK,), jnp.int32),
            pltpu.VMEM((_BLK,), jnp.int32),
            pltpu.VMEM((_BLK,), jnp.int32),
            pltpu.VMEM((_BLK,), jnp.int32),
            pltpu.VMEM((_BLK, D), jnp.float32),
            pltpu.VMEM((_BLK, D), jnp.float32),
            pltpu.VMEM((_BLK, D), jnp.float32),
            pltpu.VMEM((_BLK, D), jnp.float32),
            pltpu.VMEM((_BLK, D), jnp.float32),
            pltpu.VMEM((_BLK, D), jnp.float32),
            pltpu.SemaphoreType.DMA,
            pltpu.SemaphoreType.DMA,
            pltpu.SemaphoreType.DMA,
            pltpu.SemaphoreType.DMA,
            pltpu.SemaphoreType.DMA,
            pltpu.SemaphoreType.DMA,
        ],
    )(_sc_body)
    return f(xs, xd, et, src, dst, zeros)


_BN = 1000
_NBN = N // _BN


def _post_body(x_ref, aggp_ref, ids_ref, wu_ref, wd_ref, wt_ref,
               out_ref, cont_ref, sums_s, counts_s):
    i = pl.program_id(0)

    @pl.when(i == 0)
    def _init():
        sums_s[...] = jnp.zeros_like(sums_s)
        counts_s[...] = jnp.zeros_like(counts_s)

    x = x_ref[...]
    agg = aggp_ref[0] + aggp_ref[1]
    new_lat = jax.nn.relu(
        jnp.dot(x, wu_ref[:D, :], precision=_DP)
        + jnp.dot(agg, wu_ref[D:, :], precision=_DP))
    out_ref[...] = (jnp.dot(x, wd_ref[:D, :], precision=_DP)
                    + jnp.dot(new_lat, wd_ref[D:, :], precision=_DP))
    ids = ids_ref[0]
    iota = lax.broadcasted_iota(jnp.int32, (G, _BN), 0)
    onehot = (ids == iota).astype(jnp.float32)
    sums_s[...] += lax.dot_general(
        onehot, new_lat, dimension_numbers=(((1,), (0,)), ((), ())),
        precision=_HI)
    counts_s[...] += jnp.sum(onehot, axis=1, keepdims=True)

    @pl.when(i == _NBN - 1)
    def _fin():
        graph_emb = sums_s[...] / jnp.maximum(counts_s[...], 1.0)
        cont_ref[...] = jnp.dot(graph_emb, wt_ref[...], precision=_DP)


def _post(x, aggp, ids_row, w_upd, w_dec, w_term):
    return pl.pallas_call(
        _post_body,
        grid=(_NBN,),
        in_specs=[pl.BlockSpec((_BN, D), lambda i: (i, 0)),
                  pl.BlockSpec((2, _BN, D), lambda i: (0, i, 0)),
                  pl.BlockSpec((1, 1, _BN), lambda i: (i, 0, 0)),
                  pl.BlockSpec((2 * D, D), lambda i: (0, 0)),
                  pl.BlockSpec((2 * D, 1), lambda i: (0, 0)),
                  pl.BlockSpec((D, 1), lambda i: (0, 0))],
        out_specs=[pl.BlockSpec((_BN, 1), lambda i: (i, 0)),
                   pl.BlockSpec((G, 1), lambda i: (0, 0))],
        out_shape=[jax.ShapeDtypeStruct((N, 1), jnp.float32),
                   jax.ShapeDtypeStruct((G, 1), jnp.float32)],
        scratch_shapes=[pltpu.VMEM((G, D), jnp.float32),
                        pltpu.VMEM((G, 1), jnp.float32)],
    )(x, aggp, ids_row, w_upd, w_dec, w_term)


def kernel(x, edge_index, edge_attr, batch_ids, W_msg, W_upd, W_dec, W_term):
    src = edge_index[0]
    dst = edge_index[1]
    ws = W_msg[:D]
    wd = W_msg[D:2 * D]
    we = W_msg[2 * D:]
    zeros = jnp.zeros((N, D), jnp.float32)
    ids_row = batch_ids.astype(jnp.int32).reshape(_NBN, 1, _BN)

    xs, xd = _node_proj(x, ws, wd)
    et = _edge_term(edge_attr, we)
    aggp = _sc_msgpass(xs, xd, et, src, dst, zeros)
    out, cont = _post(x, aggp, ids_row, W_upd, W_dec, W_term)
    return out, cont.reshape(-1)
